# baseline (speedup 1.0000x reference)
import functools

import jax
import jax.numpy as jnp
from jax import lax
from jax.experimental import pallas as pl

N_MAX_OBJECTS = 1000
THRESHOLD_SCORE = 0.5
AREA_LIMIT = 1024
ROWS, LANES = 160, 128
NPAD = ROWS * LANES


def _shift_right(x, n, axis):
    pad = [(0, 0)] * x.ndim
    pad[axis] = (n, 0)
    zshape = list(x.shape)
    zshape[axis] = n
    z = jnp.zeros(zshape, dtype=x.dtype)
    sl = [slice(None)] * x.ndim
    sl[axis] = slice(0, x.shape[axis] - n)
    return jnp.concatenate([z, x[tuple(sl)]], axis=axis)


def _cumsum_log(x, axis):
    n = x.shape[axis]
    s = 1
    while s < n:
        x = x + _shift_right(x, s, axis)
        s *= 2
    return x


def _front_kernel(scores_ref, labels_ref, classes_ref, out_ref, *, n_classes):
    s = scores_ref[...]
    labels = labels_ref[...]
    key = lax.bitcast_convert_type(s, jnp.int32)

    k = N_MAX_OBJECTS

    def body(_, carry):
        lo, hi = carry
        mid = (lo + hi) // 2
        cnt = jnp.sum((key >= mid).astype(jnp.int32))
        take = cnt >= k
        return (jnp.where(take, mid, lo), jnp.where(take, hi, mid))

    lo, hi = lax.fori_loop(0, 31, body, (jnp.int32(0), jnp.int32(1 << 30)))
    thresh = lo

    gt = key > thresh
    eq = key == thresh
    cnt_gt = jnp.sum(gt.astype(jnp.int32))
    k_rem = k - cnt_gt

    eqi = eq.astype(jnp.int32)
    row_incl = _cumsum_log(eqi, axis=1)
    row_tot = row_incl[:, LANES - 1:LANES]
    row_excl_prefix = _cumsum_log(row_tot, axis=0) - row_tot
    prefix_excl = row_excl_prefix + row_incl - eqi
    topk = gt | (eq & (prefix_excl < k_rem))

    lbl_ok = jnp.zeros(labels.shape, dtype=jnp.bool_)
    for i in range(n_classes):
        lbl_ok = lbl_ok | (labels == classes_ref[0, i])

    front = topk & (s > THRESHOLD_SCORE) & lbl_ok
    out_ref[...] = front.astype(jnp.int32)


def _area_kernel(masks_ref, front_ref, out_ref):
    m = masks_ref[...]
    nz = jnp.sum((m != 0).astype(jnp.int32), axis=(1, 2), keepdims=True)
    out_ref[...] = jnp.where(
        nz > AREA_LIMIT, front_ref[...], jnp.zeros_like(front_ref)
    )


def kernel(labels, scores, masks, all_classes):
    n = scores.shape[0]
    n_classes = all_classes.shape[0]
    _, h, w = masks.shape

    scores2d = jnp.concatenate(
        [scores, jnp.zeros((NPAD - n,), jnp.float32)]).reshape(ROWS, LANES)
    labels2d = jnp.concatenate(
        [labels, jnp.full((NPAD - n,), -1, jnp.int32)]).reshape(ROWS, LANES)
    classes2d = jnp.full((8, LANES), -2, jnp.int32).at[0, :n_classes].set(
        all_classes)

    front2d = pl.pallas_call(
        functools.partial(_front_kernel, n_classes=n_classes),
        out_shape=jax.ShapeDtypeStruct((ROWS, LANES), jnp.int32),
    )(scores2d, labels2d, classes2d)
    front_col = front2d.reshape(NPAD)[:n].reshape(n, 1, 1)

    bn = 400
    grid = n // bn
    out = pl.pallas_call(
        _area_kernel,
        grid=(grid,),
        in_specs=[
            pl.BlockSpec((bn, h, w), lambda i: (i, 0, 0)),
            pl.BlockSpec((bn, 1, 1), lambda i: (i, 0, 0)),
        ],
        out_specs=pl.BlockSpec((bn, 1, 1), lambda i: (i, 0, 0)),
        out_shape=jax.ShapeDtypeStruct((n, 1, 1), jnp.int32),
    )(masks, front_col)
    return out.reshape(n).astype(jnp.bool_)


# speedup vs baseline: 1.2887x; 1.2887x over previous
import functools

import jax
import jax.numpy as jnp
from jax import lax
from jax.experimental import pallas as pl

N_MAX_OBJECTS = 1000
THRESHOLD_SCORE = 0.5
AREA_LIMIT = 1024
ROWS, LANES = 160, 128
NPAD = ROWS * LANES


def _shift_right(x, n, axis):
    pad = [(0, 0)] * x.ndim
    pad[axis] = (n, 0)
    zshape = list(x.shape)
    zshape[axis] = n
    z = jnp.zeros(zshape, dtype=x.dtype)
    sl = [slice(None)] * x.ndim
    sl[axis] = slice(0, x.shape[axis] - n)
    return jnp.concatenate([z, x[tuple(sl)]], axis=axis)


def _cumsum_log(x, axis):
    n = x.shape[axis]
    s = 1
    while s < n:
        x = x + _shift_right(x, s, axis)
        s *= 2
    return x


def _front_kernel(scores_ref, labels_ref, classes_ref, out_ref, *, n_classes):
    s = scores_ref[...]
    labels = labels_ref[...]
    key = lax.bitcast_convert_type(s, jnp.int32)

    k = N_MAX_OBJECTS

    def body(_, carry):
        lo, hi = carry
        mid = (lo + hi) // 2
        cnt = jnp.sum((key >= mid).astype(jnp.int32))
        take = cnt >= k
        return (jnp.where(take, mid, lo), jnp.where(take, hi, mid))

    lo, hi = lax.fori_loop(0, 31, body, (jnp.int32(0), jnp.int32(1 << 30)))
    thresh = lo

    gt = key > thresh
    eq = key == thresh
    cnt_gt = jnp.sum(gt.astype(jnp.int32))
    k_rem = k - cnt_gt

    eqi = eq.astype(jnp.int32)
    row_incl = _cumsum_log(eqi, axis=1)
    row_tot = row_incl[:, LANES - 1:LANES]
    row_excl_prefix = _cumsum_log(row_tot, axis=0) - row_tot
    prefix_excl = row_excl_prefix + row_incl - eqi
    topk = gt | (eq & (prefix_excl < k_rem))

    lbl_ok = jnp.zeros(labels.shape, dtype=jnp.bool_)
    for i in range(n_classes):
        lbl_ok = lbl_ok | (labels == classes_ref[0, i])

    front = topk & (s > THRESHOLD_SCORE) & lbl_ok
    out_ref[...] = front.astype(jnp.int32)


def _area_kernel(masks_ref, front_ref, out_ref, *, bn, rows_per_det):
    m = masks_ref[...]
    nzi = (m != 0).astype(jnp.int32).reshape(bn, rows_per_det, 128)
    nz = jnp.sum(nzi, axis=(1, 2), keepdims=True)[:, :, 0]
    out_ref[...] = jnp.where(
        nz > AREA_LIMIT, front_ref[...], jnp.zeros_like(front_ref)
    )


def kernel(labels, scores, masks, all_classes):
    n = scores.shape[0]
    n_classes = all_classes.shape[0]
    _, h, w = masks.shape

    scores2d = jnp.concatenate(
        [scores, jnp.zeros((NPAD - n,), jnp.float32)]).reshape(ROWS, LANES)
    labels2d = jnp.concatenate(
        [labels, jnp.full((NPAD - n,), -1, jnp.int32)]).reshape(ROWS, LANES)
    classes2d = jnp.full((8, LANES), -2, jnp.int32).at[0, :n_classes].set(
        all_classes)

    front2d = pl.pallas_call(
        functools.partial(_front_kernel, n_classes=n_classes),
        out_shape=jax.ShapeDtypeStruct((ROWS, LANES), jnp.int32),
    )(scores2d, labels2d, classes2d)
    front_col = front2d.reshape(NPAD)[:n].reshape(n, 1)

    bn = 400
    grid = n // bn
    rows_per_det = h * w // 128
    masks_flat = masks.reshape(n * rows_per_det, 128)
    out = pl.pallas_call(
        functools.partial(_area_kernel, bn=bn, rows_per_det=rows_per_det),
        grid=(grid,),
        in_specs=[
            pl.BlockSpec((bn * rows_per_det, 128), lambda i: (i, 0)),
            pl.BlockSpec((bn, 1), lambda i: (i, 0)),
        ],
        out_specs=pl.BlockSpec((bn, 1), lambda i: (i, 0)),
        out_shape=jax.ShapeDtypeStruct((n, 1), jnp.int32),
    )(masks_flat, front_col)
    return out.reshape(n).astype(jnp.bool_)


# speedup vs baseline: 10.5598x; 8.1939x over previous
import functools

import jax
import jax.numpy as jnp
from jax import lax
from jax.experimental import pallas as pl
from jax.experimental.pallas import tpu as pltpu

N_MAX_OBJECTS = 1000
THRESHOLD_SCORE = 0.5
AREA_LIMIT = 1024
ROWS, LANES = 160, 128
NPAD = ROWS * LANES


def _shift_right(x, n, axis):
    pad = [(0, 0)] * x.ndim
    pad[axis] = (n, 0)
    zshape = list(x.shape)
    zshape[axis] = n
    z = jnp.zeros(zshape, dtype=x.dtype)
    sl = [slice(None)] * x.ndim
    sl[axis] = slice(0, x.shape[axis] - n)
    return jnp.concatenate([z, x[tuple(sl)]], axis=axis)


def _cumsum_log(x, axis):
    n = x.shape[axis]
    s = 1
    while s < n:
        x = x + _shift_right(x, s, axis)
        s *= 2
    return x


def _front_kernel(scores_ref, labels_ref, classes_ref, out_ref, *, n_classes):
    s = scores_ref[...]
    labels = labels_ref[...]
    key = lax.bitcast_convert_type(s, jnp.int32)

    k = N_MAX_OBJECTS

    def body(_, carry):
        lo, hi = carry
        mid = (lo + hi) // 2
        cnt = jnp.sum((key >= mid).astype(jnp.int32))
        take = cnt >= k
        return (jnp.where(take, mid, lo), jnp.where(take, hi, mid))

    lo, hi = lax.fori_loop(0, 31, body, (jnp.int32(0), jnp.int32(1 << 30)))
    thresh = lo

    gt = key > thresh
    eq = key == thresh
    cnt_gt = jnp.sum(gt.astype(jnp.int32))
    k_rem = k - cnt_gt

    eqi = eq.astype(jnp.int32)
    row_incl = _cumsum_log(eqi, axis=1)
    row_tot = row_incl[:, LANES - 1:LANES]
    row_excl_prefix = _cumsum_log(row_tot, axis=0) - row_tot
    prefix_excl = row_excl_prefix + row_incl - eqi
    topk = gt | (eq & (prefix_excl < k_rem))

    lbl_ok = jnp.zeros(labels.shape, dtype=jnp.bool_)
    for i in range(n_classes):
        lbl_ok = lbl_ok | (labels == classes_ref[0, i])

    front = topk & (s > THRESHOLD_SCORE) & lbl_ok
    out_ref[...] = front.astype(jnp.int32)


def _area_kernel(mt_ref, out_ref, acc_ref, *, n_steps, br):
    step = pl.program_id(0)

    @pl.when(step == 0)
    def _init():
        acc_ref[...] = jnp.zeros_like(acc_ref)

    m = mt_ref[...]
    nz = (m != 0).astype(jnp.int32).reshape(br // 8, 8, m.shape[1])
    acc_ref[...] += jnp.sum(nz, axis=0)

    @pl.when(step == n_steps - 1)
    def _fin():
        total = jnp.sum(acc_ref[...], axis=0, keepdims=True)
        out_ref[...] = (total > AREA_LIMIT).astype(jnp.int32)


def kernel(labels, scores, masks, all_classes):
    n = scores.shape[0]
    n_classes = all_classes.shape[0]
    _, h, w = masks.shape
    hw = h * w

    scores2d = jnp.concatenate(
        [scores, jnp.zeros((NPAD - n,), jnp.float32)]).reshape(ROWS, LANES)
    labels2d = jnp.concatenate(
        [labels, jnp.full((NPAD - n,), -1, jnp.int32)]).reshape(ROWS, LANES)
    classes2d = jnp.full((8, LANES), -2, jnp.int32).at[0, :n_classes].set(
        all_classes)

    front2d = pl.pallas_call(
        functools.partial(_front_kernel, n_classes=n_classes),
        out_shape=jax.ShapeDtypeStruct((ROWS, LANES), jnp.int32),
    )(scores2d, labels2d, classes2d)
    front = front2d.reshape(NPAD)[:n] > 0

    masks_t = jnp.transpose(masks, (1, 2, 0)).reshape(hw, n)
    br = 256
    n_steps = hw // br
    area2d = pl.pallas_call(
        functools.partial(_area_kernel, n_steps=n_steps, br=br),
        grid=(n_steps,),
        in_specs=[pl.BlockSpec((br, n), lambda i: (i, 0))],
        out_specs=pl.BlockSpec((1, n), lambda i: (0, 0)),
        out_shape=jax.ShapeDtypeStruct((1, n), jnp.int32),
        scratch_shapes=[pltpu.VMEM((8, n), jnp.int32)],
    )(masks_t)
    return front & (area2d.reshape(n) > 0)
